# baseline (device time: 126948 ns/iter reference)
import jax
import jax.numpy as jnp
from jax import lax
from jax.experimental import pallas as pl
from jax.experimental.pallas import tpu as pltpu

N_DEV = 4
SQ = 1024
SKV = 1024
D = 1024
HQ_LOCAL = 8
DH = 128
SCALE = 0.08838834764831843
CHUNK = SQ // N_DEV
NEG = -1e9


def _body(x_ref, wq_ref, k_ref, v_ref, wo_ref, out_ref,
          acc_ref, rs_buf, rs_send_sems, rs_recv_sems,
          ag_send_sems, ag_recv_sems):
    my = lax.axis_index("i")
    left = lax.rem(my + N_DEV - 1, N_DEV)
    right = lax.rem(my + 1, N_DEV)

    barrier = pltpu.get_barrier_semaphore()
    for nbr in (left, right):
        pl.semaphore_signal(barrier, inc=1, device_id=(nbr,),
                            device_id_type=pl.DeviceIdType.MESH)
    pl.semaphore_wait(barrier, 2)

    q = jnp.dot(x_ref[...], wq_ref[...], preferred_element_type=jnp.float32)
    q = q.astype(jnp.bfloat16)

    qb = lax.broadcasted_iota(jnp.int32, (SQ, SKV), 0) // 64
    kb = lax.broadcasted_iota(jnp.int32, (SQ, SKV), 1) // 64
    mask = (qb == kb) | (kb == 0) | (lax.rem(qb + kb, 3) == 0)

    for h in range(HQ_LOCAL):
        qh = q[:, h * DH:(h + 1) * DH]
        s = lax.dot_general(qh, k_ref[h], (((1,), (1,)), ((), ())),
                            preferred_element_type=jnp.float32) * SCALE
        s = jnp.where(mask, s, NEG)
        m = jnp.max(s, axis=-1, keepdims=True)
        w = jnp.exp(s - m)
        w = (w / jnp.sum(w, axis=-1, keepdims=True)).astype(jnp.bfloat16)
        ctx_h = jnp.dot(w, v_ref[h], preferred_element_type=jnp.float32)
        ph = jnp.dot(ctx_h.astype(jnp.bfloat16),
                     wo_ref[h * DH:(h + 1) * DH, :],
                     preferred_element_type=jnp.float32)
        if h == 0:
            acc_ref[...] = ph
        else:
            acc_ref[...] = acc_ref[...] + ph

    for h in range(N_DEV - 1):
        s_idx = lax.rem(my - h + N_DEV, N_DEV)
        r_idx = lax.rem(my - h - 1 + N_DEV, N_DEV)
        rdma = pltpu.make_async_remote_copy(
            src_ref=acc_ref.at[pl.ds(s_idx * CHUNK, CHUNK)],
            dst_ref=rs_buf.at[h],
            send_sem=rs_send_sems.at[h],
            recv_sem=rs_recv_sems.at[h],
            device_id=(right,),
            device_id_type=pl.DeviceIdType.MESH,
        )
        rdma.start()
        rdma.wait()
        acc_ref[pl.ds(r_idx * CHUNK, CHUNK), :] = (
            acc_ref[pl.ds(r_idx * CHUNK, CHUNK), :] + rs_buf[h]
        )

    own = lax.rem(my + 1, N_DEV)
    out_ref[pl.ds(own * CHUNK, CHUNK), :] = acc_ref[pl.ds(own * CHUNK, CHUNK), :]
    for h in range(N_DEV - 1):
        g = lax.rem(own - h + N_DEV, N_DEV)
        rdma = pltpu.make_async_remote_copy(
            src_ref=out_ref.at[pl.ds(g * CHUNK, CHUNK)],
            dst_ref=out_ref.at[pl.ds(g * CHUNK, CHUNK)],
            send_sem=ag_send_sems.at[h],
            recv_sem=ag_recv_sems.at[h],
            device_id=(right,),
            device_id_type=pl.DeviceIdType.MESH,
        )
        rdma.start()
        rdma.wait()


def kernel(x, Wq, K_ext, V_ext, Wo):
    my = lax.axis_index("i")
    x2 = x.reshape(SQ, D).astype(jnp.bfloat16)
    Wq = Wq.astype(jnp.bfloat16)
    Wo = Wo.astype(jnp.bfloat16)
    k_loc = jnp.transpose(
        lax.dynamic_slice_in_dim(K_ext[0], my * HQ_LOCAL, HQ_LOCAL, axis=1),
        (1, 0, 2)).astype(jnp.bfloat16)
    v_loc = jnp.transpose(
        lax.dynamic_slice_in_dim(V_ext[0], my * HQ_LOCAL, HQ_LOCAL, axis=1),
        (1, 0, 2)).astype(jnp.bfloat16)

    out = pl.pallas_call(
        _body,
        out_shape=jax.ShapeDtypeStruct((SQ, D), jnp.float32),
        in_specs=[pl.BlockSpec(memory_space=pltpu.VMEM)] * 5,
        out_specs=pl.BlockSpec(memory_space=pltpu.VMEM),
        scratch_shapes=[
            pltpu.VMEM((SQ, D), jnp.float32),
            pltpu.VMEM((N_DEV - 1, CHUNK, D), jnp.float32),
            pltpu.SemaphoreType.DMA((N_DEV - 1,)),
            pltpu.SemaphoreType.DMA((N_DEV - 1,)),
            pltpu.SemaphoreType.DMA((N_DEV - 1,)),
            pltpu.SemaphoreType.DMA((N_DEV - 1,)),
        ],
        compiler_params=pltpu.CompilerParams(collective_id=0),
    )(x2, Wq, k_loc, v_loc, Wo)
    return out.reshape(1, SQ, D)


# device time: 68240 ns/iter; 1.8603x vs baseline; 1.8603x over previous
import jax
import jax.numpy as jnp
from jax import lax
from jax.experimental import pallas as pl
from jax.experimental.pallas import tpu as pltpu

N_DEV = 4
SQ = 1024
SKV = 1024
D = 1024
HQ_LOCAL = 8
DH = 128
SCALE = 0.08838834764831843
CH = SQ // (2 * N_DEV)
NEG = -1e5


def _compute_piece(x_ref, wq_ref, k_ref, v_ref, wo_ref, acc_ref, o):
    qc = jnp.dot(x_ref[pl.ds(o, CH), :], wq_ref[...],
                 preferred_element_type=jnp.float32).astype(jnp.bfloat16)
    row = o + lax.broadcasted_iota(jnp.int32, (CH, SKV), 0)
    col = lax.broadcasted_iota(jnp.int32, (CH, SKV), 1)
    qb = row // 64
    kb = col // 64
    mask = (qb == kb) | (kb == 0) | (lax.rem(qb + kb, 3) == 0)
    bias = jnp.where(mask, 0.0, NEG)
    ctxs = []
    for h in range(HQ_LOCAL):
        s = lax.dot_general(qc[:, h * DH:(h + 1) * DH], k_ref[h],
                            (((1,), (1,)), ((), ())),
                            preferred_element_type=jnp.float32)
        w = jnp.exp(s * SCALE + bias)
        wsum = jnp.sum(w, axis=-1, keepdims=True)
        ctx = jnp.dot(w.astype(jnp.bfloat16), v_ref[h],
                      preferred_element_type=jnp.float32)
        ctxs.append((ctx / wsum).astype(jnp.bfloat16))
    ctx_c = jnp.concatenate(ctxs, axis=1)
    acc_ref[pl.ds(o, CH), :] = jnp.dot(ctx_c, wo_ref[...],
                                       preferred_element_type=jnp.float32)


def _body(x_ref, wq_ref, k_ref, v_ref, wo_ref, out_ref, acc_ref,
          rs_buf_a, rs_buf_b,
          rs_a_ssem, rs_a_rsem, rs_b_ssem, rs_b_rsem,
          ag_a_ssem, ag_a_rsem, ag_b_ssem, ag_b_rsem):
    my = lax.axis_index("i")
    left = lax.rem(my + N_DEV - 1, N_DEV)
    right = lax.rem(my + 1, N_DEV)

    barrier = pltpu.get_barrier_semaphore()
    for nbr in (left, right):
        pl.semaphore_signal(barrier, inc=1, device_id=(nbr,),
                            device_id_type=pl.DeviceIdType.MESH)
    pl.semaphore_wait(barrier, 2)

    pending = []

    def _send(src_off, dst_ref, ssem, rsem, h, nbr):
        rdma = pltpu.make_async_remote_copy(
            src_ref=acc_ref.at[pl.ds(src_off, CH)],
            dst_ref=dst_ref,
            send_sem=ssem.at[h], recv_sem=rsem.at[h],
            device_id=(nbr,), device_id_type=pl.DeviceIdType.MESH)
        rdma.start()
        pending.append(rdma)
        return rdma

    rs_a, rs_b = [], []
    for k in range(N_DEV):
        a_k = lax.rem(my - k + N_DEV, N_DEV)
        b_k = lax.rem(my + k, N_DEV)
        oa = a_k * CH
        ob = (b_k + N_DEV) * CH
        _compute_piece(x_ref, wq_ref, k_ref, v_ref, wo_ref, acc_ref, oa)
        _compute_piece(x_ref, wq_ref, k_ref, v_ref, wo_ref, acc_ref, ob)
        if k > 0:
            rs_a[k - 1].wait_recv()
            acc_ref[pl.ds(oa, CH), :] = (
                acc_ref[pl.ds(oa, CH), :] + rs_buf_a[k - 1])
            rs_b[k - 1].wait_recv()
            acc_ref[pl.ds(ob, CH), :] = (
                acc_ref[pl.ds(ob, CH), :] + rs_buf_b[k - 1])
        if k < N_DEV - 1:
            rs_a.append(_send(oa, rs_buf_a.at[k], rs_a_ssem, rs_a_rsem,
                              k, right))
            rs_b.append(_send(ob, rs_buf_b.at[k], rs_b_ssem, rs_b_rsem,
                              k, left))

    own_a = lax.rem(my + 1, N_DEV)
    own_b = lax.rem(my + N_DEV - 1, N_DEV)
    oa = own_a * CH
    ob = (own_b + N_DEV) * CH
    out_ref[pl.ds(oa, CH), :] = acc_ref[pl.ds(oa, CH), :]
    out_ref[pl.ds(ob, CH), :] = acc_ref[pl.ds(ob, CH), :]

    def _ag_hop(g_off, ssem, rsem, h, nbr):
        rdma = pltpu.make_async_remote_copy(
            src_ref=out_ref.at[pl.ds(g_off, CH)],
            dst_ref=out_ref.at[pl.ds(g_off, CH)],
            send_sem=ssem.at[h], recv_sem=rsem.at[h],
            device_id=(nbr,), device_id_type=pl.DeviceIdType.MESH)
        rdma.start()
        pending.append(rdma)

    def _ag_recv(r_off, ssem, rsem, h, nbr):
        return pltpu.make_async_remote_copy(
            src_ref=out_ref.at[pl.ds(r_off, CH)],
            dst_ref=out_ref.at[pl.ds(r_off, CH)],
            send_sem=ssem.at[h], recv_sem=rsem.at[h],
            device_id=(nbr,), device_id_type=pl.DeviceIdType.MESH)

    ag_a, ag_b = [], []
    for h in range(N_DEV - 1):
        ga = lax.rem(own_a - h + N_DEV, N_DEV)
        gb = lax.rem(own_b + h, N_DEV)
        ra = lax.rem(own_a - h - 1 + N_DEV, N_DEV)
        rb = lax.rem(own_b + h + 1, N_DEV)
        if h > 0:
            ag_a[h - 1].wait_recv()
            ag_b[h - 1].wait_recv()
        _ag_hop(ga * CH, ag_a_ssem, ag_a_rsem, h, right)
        _ag_hop((gb + N_DEV) * CH, ag_b_ssem, ag_b_rsem, h, left)
        ag_a.append(_ag_recv(ra * CH, ag_a_ssem, ag_a_rsem, h, left))
        ag_b.append(_ag_recv((rb + N_DEV) * CH, ag_b_ssem, ag_b_rsem,
                             h, right))
    ag_a[-1].wait_recv()
    ag_b[-1].wait_recv()
    for d in pending:
        d.wait_send()


def kernel(x, Wq, K_ext, V_ext, Wo):
    my = lax.axis_index("i")
    x2 = x.reshape(SQ, D).astype(jnp.bfloat16)
    Wq = Wq.astype(jnp.bfloat16)
    Wo = Wo.astype(jnp.bfloat16)
    k_loc = jnp.transpose(
        lax.dynamic_slice_in_dim(K_ext[0], my * HQ_LOCAL, HQ_LOCAL, axis=1),
        (1, 0, 2)).astype(jnp.bfloat16)
    v_loc = jnp.transpose(
        lax.dynamic_slice_in_dim(V_ext[0], my * HQ_LOCAL, HQ_LOCAL, axis=1),
        (1, 0, 2)).astype(jnp.bfloat16)

    out = pl.pallas_call(
        _body,
        out_shape=jax.ShapeDtypeStruct((SQ, D), jnp.float32),
        in_specs=[pl.BlockSpec(memory_space=pltpu.VMEM)] * 5,
        out_specs=pl.BlockSpec(memory_space=pltpu.VMEM),
        scratch_shapes=[
            pltpu.VMEM((SQ, D), jnp.float32),
            pltpu.VMEM((N_DEV - 1, CH, D), jnp.float32),
            pltpu.VMEM((N_DEV - 1, CH, D), jnp.float32),
            pltpu.SemaphoreType.DMA((N_DEV - 1,)),
            pltpu.SemaphoreType.DMA((N_DEV - 1,)),
            pltpu.SemaphoreType.DMA((N_DEV - 1,)),
            pltpu.SemaphoreType.DMA((N_DEV - 1,)),
            pltpu.SemaphoreType.DMA((N_DEV - 1,)),
            pltpu.SemaphoreType.DMA((N_DEV - 1,)),
            pltpu.SemaphoreType.DMA((N_DEV - 1,)),
            pltpu.SemaphoreType.DMA((N_DEV - 1,)),
        ],
        compiler_params=pltpu.CompilerParams(collective_id=0),
    )(x2, Wq, k_loc, v_loc, Wo)
    return out.reshape(1, SQ, D)


# device time: 56967 ns/iter; 2.2284x vs baseline; 1.1979x over previous
import jax
import jax.numpy as jnp
from jax import lax
from jax.experimental import pallas as pl
from jax.experimental.pallas import tpu as pltpu

N_DEV = 4
SQ = 1024
SKV = 1024
D = 1024
HQ_LOCAL = 8
DH = 128
SCALE = 0.08838834764831843
CH = SQ // (2 * N_DEV)
NEG = -1e5


def _compute_piece(x_ref, wq_ref, k_ref, v_ref, wo_ref, acc_ref, o):
    qc = jnp.dot(x_ref[pl.ds(o, CH), :], wq_ref[...],
                 preferred_element_type=jnp.float32).astype(jnp.bfloat16)
    row = o + lax.broadcasted_iota(jnp.int32, (CH, SKV), 0)
    col = lax.broadcasted_iota(jnp.int32, (CH, SKV), 1)
    qb = row // 64
    kb = col // 64
    mask = (qb == kb) | (kb == 0) | (lax.rem(qb + kb, 3) == 0)
    bias = jnp.where(mask, 0.0, NEG)
    ctxs = []
    for h in range(HQ_LOCAL):
        s = lax.dot_general(qc[:, h * DH:(h + 1) * DH], k_ref[h],
                            (((1,), (1,)), ((), ())),
                            preferred_element_type=jnp.float32)
        w = jnp.exp(s * SCALE + bias)
        wsum = jnp.sum(w, axis=-1, keepdims=True)
        ctx = jnp.dot(w.astype(jnp.bfloat16), v_ref[h],
                      preferred_element_type=jnp.float32)
        ctxs.append((ctx / wsum).astype(jnp.bfloat16))
    ctx_c = jnp.concatenate(ctxs, axis=1)
    acc_ref[pl.ds(o, CH), :] = jnp.dot(
        ctx_c, wo_ref[...],
        preferred_element_type=jnp.float32).astype(jnp.bfloat16)


def _body(x_ref, wq_ref, k_ref, v_ref, wo_ref, out_ref, acc_ref, gath_ref,
          rs_buf_a, rs_buf_b,
          rs_a_ssem, rs_a_rsem, rs_b_ssem, rs_b_rsem,
          ag_a_ssem, ag_a_rsem, ag_b_ssem, ag_b_rsem):
    my = lax.axis_index("i")
    left = lax.rem(my + N_DEV - 1, N_DEV)
    right = lax.rem(my + 1, N_DEV)

    barrier = pltpu.get_barrier_semaphore()
    for nbr in (left, right):
        pl.semaphore_signal(barrier, inc=1, device_id=(nbr,),
                            device_id_type=pl.DeviceIdType.MESH)
    pl.semaphore_wait(barrier, 2)

    pending = []

    def _send(src_off, dst_ref, ssem, rsem, h, nbr):
        rdma = pltpu.make_async_remote_copy(
            src_ref=acc_ref.at[pl.ds(src_off, CH)],
            dst_ref=dst_ref,
            send_sem=ssem.at[h], recv_sem=rsem.at[h],
            device_id=(nbr,), device_id_type=pl.DeviceIdType.MESH)
        rdma.start()
        pending.append(rdma)
        return rdma

    rs_a, rs_b = [], []
    for k in range(N_DEV):
        a_k = lax.rem(my - k + N_DEV, N_DEV)
        b_k = lax.rem(my + k, N_DEV)
        oa = a_k * CH
        ob = (b_k + N_DEV) * CH
        _compute_piece(x_ref, wq_ref, k_ref, v_ref, wo_ref, acc_ref, oa)
        _compute_piece(x_ref, wq_ref, k_ref, v_ref, wo_ref, acc_ref, ob)
        if k > 0:
            rs_a[k - 1].wait_recv()
            acc_ref[pl.ds(oa, CH), :] = (
                acc_ref[pl.ds(oa, CH), :] + rs_buf_a[k - 1])
            rs_b[k - 1].wait_recv()
            acc_ref[pl.ds(ob, CH), :] = (
                acc_ref[pl.ds(ob, CH), :] + rs_buf_b[k - 1])
        if k < N_DEV - 1:
            rs_a.append(_send(oa, rs_buf_a.at[k], rs_a_ssem, rs_a_rsem,
                              k, right))
            rs_b.append(_send(ob, rs_buf_b.at[k], rs_b_ssem, rs_b_rsem,
                              k, left))

    own_a = lax.rem(my + 1, N_DEV)
    own_b = lax.rem(my + N_DEV - 1, N_DEV)
    oa = own_a * CH
    ob = (own_b + N_DEV) * CH
    gath_ref[pl.ds(oa, CH), :] = acc_ref[pl.ds(oa, CH), :]
    gath_ref[pl.ds(ob, CH), :] = acc_ref[pl.ds(ob, CH), :]

    def _ag_hop(g_off, ssem, rsem, h, nbr):
        rdma = pltpu.make_async_remote_copy(
            src_ref=gath_ref.at[pl.ds(g_off, CH)],
            dst_ref=gath_ref.at[pl.ds(g_off, CH)],
            send_sem=ssem.at[h], recv_sem=rsem.at[h],
            device_id=(nbr,), device_id_type=pl.DeviceIdType.MESH)
        rdma.start()
        pending.append(rdma)

    def _ag_recv(r_off, ssem, rsem, h, nbr):
        return pltpu.make_async_remote_copy(
            src_ref=gath_ref.at[pl.ds(r_off, CH)],
            dst_ref=gath_ref.at[pl.ds(r_off, CH)],
            send_sem=ssem.at[h], recv_sem=rsem.at[h],
            device_id=(nbr,), device_id_type=pl.DeviceIdType.MESH)

    def _cast_out(off):
        out_ref[pl.ds(off, CH), :] = gath_ref[pl.ds(off, CH), :].astype(
            jnp.float32)

    ag_a, ag_b = [], []
    for h in range(N_DEV - 1):
        ga = lax.rem(own_a - h + N_DEV, N_DEV)
        gb = lax.rem(own_b + h, N_DEV)
        ra = lax.rem(own_a - h - 1 + N_DEV, N_DEV)
        rb = lax.rem(own_b + h + 1, N_DEV)
        if h > 0:
            ag_a[h - 1].wait_recv()
            ag_b[h - 1].wait_recv()
        _ag_hop(ga * CH, ag_a_ssem, ag_a_rsem, h, right)
        _ag_hop((gb + N_DEV) * CH, ag_b_ssem, ag_b_rsem, h, left)
        if h > 0:
            _cast_out(ga * CH)
            _cast_out((gb + N_DEV) * CH)
        ag_a.append(_ag_recv(ra * CH, ag_a_ssem, ag_a_rsem, h, left))
        ag_b.append(_ag_recv((rb + N_DEV) * CH, ag_b_ssem, ag_b_rsem,
                             h, right))
    _cast_out(oa)
    _cast_out(ob)
    ag_a[-1].wait_recv()
    ag_b[-1].wait_recv()
    _cast_out(lax.rem(own_a - N_DEV + 1 + N_DEV, N_DEV) * CH)
    _cast_out((lax.rem(own_b + N_DEV - 1, N_DEV) + N_DEV) * CH)
    for d in pending:
        d.wait_send()


def kernel(x, Wq, K_ext, V_ext, Wo):
    my = lax.axis_index("i")
    x2 = x.reshape(SQ, D).astype(jnp.bfloat16)
    Wq = Wq.astype(jnp.bfloat16)
    Wo = Wo.astype(jnp.bfloat16)
    k_loc = jnp.transpose(
        lax.dynamic_slice_in_dim(K_ext[0], my * HQ_LOCAL, HQ_LOCAL, axis=1),
        (1, 0, 2)).astype(jnp.bfloat16)
    v_loc = jnp.transpose(
        lax.dynamic_slice_in_dim(V_ext[0], my * HQ_LOCAL, HQ_LOCAL, axis=1),
        (1, 0, 2)).astype(jnp.bfloat16)

    out = pl.pallas_call(
        _body,
        out_shape=jax.ShapeDtypeStruct((SQ, D), jnp.float32),
        in_specs=[pl.BlockSpec(memory_space=pltpu.VMEM)] * 5,
        out_specs=pl.BlockSpec(memory_space=pltpu.VMEM),
        scratch_shapes=[
            pltpu.VMEM((SQ, D), jnp.bfloat16),
            pltpu.VMEM((SQ, D), jnp.bfloat16),
            pltpu.VMEM((N_DEV - 1, CH, D), jnp.bfloat16),
            pltpu.VMEM((N_DEV - 1, CH, D), jnp.bfloat16),
            pltpu.SemaphoreType.DMA((N_DEV - 1,)),
            pltpu.SemaphoreType.DMA((N_DEV - 1,)),
            pltpu.SemaphoreType.DMA((N_DEV - 1,)),
            pltpu.SemaphoreType.DMA((N_DEV - 1,)),
            pltpu.SemaphoreType.DMA((N_DEV - 1,)),
            pltpu.SemaphoreType.DMA((N_DEV - 1,)),
            pltpu.SemaphoreType.DMA((N_DEV - 1,)),
            pltpu.SemaphoreType.DMA((N_DEV - 1,)),
        ],
        compiler_params=pltpu.CompilerParams(collective_id=0),
    )(x2, Wq, k_loc, v_loc, Wo)
    return out.reshape(1, SQ, D)


# device time: 43259 ns/iter; 2.9346x vs baseline; 1.3169x over previous
import jax
import jax.numpy as jnp
from jax import lax
from jax.experimental import pallas as pl
from jax.experimental.pallas import tpu as pltpu

N_DEV = 4
SQ = 1024
SKV = 1024
D = 1024
HQ_LOCAL = 8
DH = 128
SCALE = 0.08838834764831843
CH = SQ // (2 * N_DEV)
SUB = 4
SUBR = CH // SUB
NEG = -1e5


def _compute_pair(x_ref, wq_ref, k_ref, v_ref, wo_ref, acc_ref, oa, ob,
                  kv_dmas=()):
    M = 2 * CH
    xx = jnp.concatenate(
        [x_ref[pl.ds(oa, CH), :], x_ref[pl.ds(ob, CH), :]], axis=0)
    qc = jnp.dot(xx, wq_ref[...], preferred_element_type=jnp.float32)
    qc = qc * SCALE
    for d in kv_dmas:
        d.wait()
    io = lax.broadcasted_iota(jnp.int32, (M, SKV), 0)
    row = jnp.where(io < CH, oa + io, ob + io - CH)
    col = lax.broadcasted_iota(jnp.int32, (M, SKV), 1)
    qb = row // 64
    kb = col // 64
    mask = (qb == kb) | (kb == 0) | (lax.rem(qb + kb, 3) == 0)
    bias = jnp.where(mask, 0.0, NEG)
    ctxs = []
    for h in range(HQ_LOCAL):
        s = lax.dot_general(qc[:, h * DH:(h + 1) * DH],
                            k_ref[:, h * DH:(h + 1) * DH],
                            (((1,), (1,)), ((), ())),
                            preferred_element_type=jnp.float32)
        w = jnp.exp(s + bias)
        wsum = jnp.sum(w, axis=-1, keepdims=True)
        ctx = jnp.dot(w, v_ref[:, h * DH:(h + 1) * DH],
                      preferred_element_type=jnp.float32)
        ctxs.append(ctx / wsum)
    ctx_c = jnp.concatenate(ctxs, axis=1)
    po = jnp.dot(ctx_c, wo_ref[...],
                 preferred_element_type=jnp.float32).astype(jnp.bfloat16)
    acc_ref[pl.ds(oa, CH), :] = po[0:CH, :]
    acc_ref[pl.ds(ob, CH), :] = po[CH:M, :]


def _body(x_ref, wq_ref, k_hbm, v_hbm, wo_ref, out_ref, acc_ref, gath_ref,
          k_f32, v_f32, rs_buf_a, rs_buf_b,
          kv_sems,
          rs_a_ssem, rs_a_rsem, rs_b_ssem, rs_b_rsem,
          ag_a_ssem, ag_a_rsem, ag_b_ssem, ag_b_rsem):
    my = lax.axis_index("i")
    left = lax.rem(my + N_DEV - 1, N_DEV)
    right = lax.rem(my + 1, N_DEV)

    kv_dmas = []
    for h in range(HQ_LOCAL):
        for j, (hbm, dst) in enumerate(((k_hbm, k_f32), (v_hbm, v_f32))):
            d = pltpu.make_async_copy(
                hbm.at[0, :, my * HQ_LOCAL + h, :],
                dst.at[:, pl.ds(h * DH, DH)],
                kv_sems.at[j, h])
            d.start()
            kv_dmas.append(d)

    barrier = pltpu.get_barrier_semaphore()
    for nbr in (left, right):
        pl.semaphore_signal(barrier, inc=1, device_id=(nbr,),
                            device_id_type=pl.DeviceIdType.MESH)
    pl.semaphore_wait(barrier, 2)

    pending = []

    def _send(src_off, dst_ref, ssem, rsem, h, nbr):
        rdma = pltpu.make_async_remote_copy(
            src_ref=acc_ref.at[pl.ds(src_off, CH)],
            dst_ref=dst_ref,
            send_sem=ssem.at[h], recv_sem=rsem.at[h],
            device_id=(nbr,), device_id_type=pl.DeviceIdType.MESH)
        rdma.start()
        pending.append(rdma)
        return rdma

    rs_a, rs_b = [], []
    for k in range(N_DEV):
        a_k = lax.rem(my - k + N_DEV, N_DEV)
        b_k = lax.rem(my + k, N_DEV)
        oa = a_k * CH
        ob = (b_k + N_DEV) * CH
        _compute_pair(x_ref, wq_ref, k_f32, v_f32, wo_ref, acc_ref, oa, ob,
                      kv_dmas if k == 0 else ())
        if k > 0:
            rs_a[k - 1].wait_recv()
            acc_ref[pl.ds(oa, CH), :] = (
                acc_ref[pl.ds(oa, CH), :] + rs_buf_a[k - 1])
        if k < N_DEV - 1:
            rs_a.append(_send(oa, rs_buf_a.at[k], rs_a_ssem, rs_a_rsem,
                              k, right))
        if k > 0:
            rs_b[k - 1].wait_recv()
            acc_ref[pl.ds(ob, CH), :] = (
                acc_ref[pl.ds(ob, CH), :] + rs_buf_b[k - 1])
        if k < N_DEV - 1:
            rs_b.append(_send(ob, rs_buf_b.at[k], rs_b_ssem, rs_b_rsem,
                              k, left))

    own_a = lax.rem(my + 1, N_DEV)
    own_b = lax.rem(my + N_DEV - 1, N_DEV)
    oa = own_a * CH
    ob = (own_b + N_DEV) * CH
    gath_ref[pl.ds(oa, CH), :] = acc_ref[pl.ds(oa, CH), :]
    gath_ref[pl.ds(ob, CH), :] = acc_ref[pl.ds(ob, CH), :]

    def _ag_hop(g_off, ssem, rsem, h, s, nbr):
        rdma = pltpu.make_async_remote_copy(
            src_ref=gath_ref.at[pl.ds(g_off + s * SUBR, SUBR)],
            dst_ref=gath_ref.at[pl.ds(g_off + s * SUBR, SUBR)],
            send_sem=ssem.at[h, s], recv_sem=rsem.at[h, s],
            device_id=(nbr,), device_id_type=pl.DeviceIdType.MESH)
        rdma.start()
        pending.append(rdma)

    def _ag_recv(r_off, ssem, rsem, h, s, nbr):
        return pltpu.make_async_remote_copy(
            src_ref=gath_ref.at[pl.ds(r_off + s * SUBR, SUBR)],
            dst_ref=gath_ref.at[pl.ds(r_off + s * SUBR, SUBR)],
            send_sem=ssem.at[h, s], recv_sem=rsem.at[h, s],
            device_id=(nbr,), device_id_type=pl.DeviceIdType.MESH)

    def _cast_out(off):
        out_ref[pl.ds(off, CH), :] = gath_ref[pl.ds(off, CH), :].astype(
            jnp.float32)

    ag_a, ag_b = [], []
    for h in range(N_DEV - 1):
        ga = lax.rem(own_a - h + N_DEV, N_DEV)
        gb = lax.rem(own_b + h, N_DEV)
        ra = lax.rem(own_a - h - 1 + N_DEV, N_DEV)
        rb = lax.rem(own_b + h + 1, N_DEV)
        suba, subb = [], []
        for s in range(SUB):
            if h > 0:
                ag_a[h - 1][s].wait_recv()
            _ag_hop(ga * CH, ag_a_ssem, ag_a_rsem, h, s, right)
            suba.append(_ag_recv(ra * CH, ag_a_ssem, ag_a_rsem, h, s, left))
            if h > 0:
                ag_b[h - 1][s].wait_recv()
            _ag_hop((gb + N_DEV) * CH, ag_b_ssem, ag_b_rsem, h, s, left)
            subb.append(_ag_recv((rb + N_DEV) * CH, ag_b_ssem, ag_b_rsem,
                                 h, s, right))
        ag_a.append(suba)
        ag_b.append(subb)
        if h > 0:
            _cast_out(ga * CH)
            _cast_out((gb + N_DEV) * CH)
    _cast_out(oa)
    _cast_out(ob)
    for s in range(SUB):
        ag_a[-1][s].wait_recv()
        ag_b[-1][s].wait_recv()
    _cast_out(lax.rem(own_a + 1, N_DEV) * CH)
    _cast_out((lax.rem(own_b + N_DEV - 1, N_DEV) + N_DEV) * CH)
    for d in pending:
        d.wait_send()


def kernel(x, Wq, K_ext, V_ext, Wo):
    x2 = x.reshape(SQ, D)

    out = pl.pallas_call(
        _body,
        out_shape=jax.ShapeDtypeStruct((SQ, D), jnp.float32),
        in_specs=[pl.BlockSpec(memory_space=pltpu.VMEM),
                  pl.BlockSpec(memory_space=pltpu.VMEM),
                  pl.BlockSpec(memory_space=pl.ANY),
                  pl.BlockSpec(memory_space=pl.ANY),
                  pl.BlockSpec(memory_space=pltpu.VMEM)],
        out_specs=pl.BlockSpec(memory_space=pltpu.VMEM),
        scratch_shapes=[
            pltpu.VMEM((SQ, D), jnp.bfloat16),
            pltpu.VMEM((SQ, D), jnp.bfloat16),
            pltpu.VMEM((SKV, D), jnp.float32),
            pltpu.VMEM((SKV, D), jnp.float32),
            pltpu.VMEM((N_DEV - 1, CH, D), jnp.bfloat16),
            pltpu.VMEM((N_DEV - 1, CH, D), jnp.bfloat16),
            pltpu.SemaphoreType.DMA((2, HQ_LOCAL)),
            pltpu.SemaphoreType.DMA((N_DEV - 1,)),
            pltpu.SemaphoreType.DMA((N_DEV - 1,)),
            pltpu.SemaphoreType.DMA((N_DEV - 1,)),
            pltpu.SemaphoreType.DMA((N_DEV - 1,)),
            pltpu.SemaphoreType.DMA((N_DEV - 1, SUB)),
            pltpu.SemaphoreType.DMA((N_DEV - 1, SUB)),
            pltpu.SemaphoreType.DMA((N_DEV - 1, SUB)),
            pltpu.SemaphoreType.DMA((N_DEV - 1, SUB)),
        ],
        compiler_params=pltpu.CompilerParams(collective_id=0),
    )(x2, Wq, K_ext, V_ext, Wo)
    return out.reshape(1, SQ, D)
